# baseline (speedup 1.0000x reference)
import functools
import itertools

import numpy as np
import jax
import jax.numpy as jnp
from jax import lax
from jax.experimental import pallas as pl
from jax.experimental.pallas import tpu as pltpu
from jax.experimental.pallas import tpu_sc as plsc

NUM_FIELDS = 26
VOCAB = 100000
EMBED_DIM = 16
ATT_VECTOR = 8
BATCH = 4096
NUM_PAIRS = NUM_FIELDS * (NUM_FIELDS - 1) // 2
P_PAD = 384
BT = 128
NUM_PLANES = NUM_FIELDS * EMBED_DIM

_row, _col = zip(*itertools.combinations(range(NUM_FIELDS), 2))
_R = np.zeros((P_PAD, NUM_FIELDS), np.float32)
_C = np.zeros((P_PAD, NUM_FIELDS), np.float32)
_R[np.arange(NUM_PAIRS), _row] = 1.0
_C[np.arange(NUM_PAIRS), _col] = 1.0


def _sc_gather(t3, sidx):
    info = plsc.get_sparse_core_info()
    nw = info.num_cores * info.num_subcores
    ppw = NUM_PLANES // nw
    mesh = plsc.VectorSubcoreMesh(core_axis_name="c", subcore_axis_name="s")

    @functools.partial(
        pl.kernel,
        out_type=jax.ShapeDtypeStruct((NUM_PLANES, BATCH), jnp.float32),
        mesh=mesh,
        scratch_types=[
            pltpu.VMEM((BATCH,), jnp.int32),
            pltpu.VMEM((ppw, BATCH), jnp.float32),
            pltpu.SemaphoreType.DMA,
            pltpu.SemaphoreType.DMA,
        ],
        compiler_params=pltpu.CompilerParams(use_tc_tiling_on_sc=False),
    )
    def gather_kernel(t_hbm, sidx_hbm, out_hbm, idx_v, vals_v, sem, gsem):
        wid = lax.axis_index("s") * info.num_cores + lax.axis_index("c")
        base = wid * ppw
        for j in range(ppw):
            plane = base + j
            f = plane // EMBED_DIM
            d = plane % EMBED_DIM
            pltpu.sync_copy(sidx_hbm.at[f], idx_v)
            pltpu.async_copy(t_hbm.at[f].at[d].at[idx_v], vals_v.at[j],
                             gsem).wait()
        pltpu.sync_copy(vals_v, out_hbm.at[pl.ds(base, ppw)])

    return gather_kernel(t3, sidx)


_SUB = 128 // BT


def _tc_body(e_ref, r_ref, c_ref, wbig_ref, wp_ref, rep_ref, wout_ref,
             batt_ref, bout_ref, o_ref):
    ed128 = e_ref[...]
    for t in range(_SUB):
        ed = ed128[:, t * BT:(t + 1) * BT]
        et = jnp.transpose(ed.reshape(NUM_FIELDS, EMBED_DIM, BT),
                           (0, 2, 1)).reshape(NUM_FIELDS, BT * EMBED_DIM)
        p = jnp.dot(r_ref[...], et, preferred_element_type=jnp.float32)
        q = jnp.dot(c_ref[...], et, preferred_element_type=jnp.float32)
        bi = p * q
        a1 = jnp.dot(bi, wbig_ref[...], preferred_element_type=jnp.float32)
        a1 = jnp.maximum(a1 + batt_ref[...], 0.0)
        logits = jnp.dot(a1, wp_ref[...], preferred_element_type=jnp.float32)
        pid = lax.broadcasted_iota(jnp.int32, logits.shape, 0)
        logits = jnp.where(pid < NUM_PAIRS, logits, -1e30)
        m = jnp.max(logits, axis=0, keepdims=True)
        ex = jnp.exp(logits - m)
        s = ex / jnp.sum(ex, axis=0, keepdims=True)
        s_exp = jnp.dot(s, rep_ref[...], preferred_element_type=jnp.float32)
        x = jnp.sum(bi * s_exp, axis=0, keepdims=True)
        y = jnp.dot(x, wout_ref[...], preferred_element_type=jnp.float32)
        y = y + bout_ref[...]
        o_ref[0, t] = (1.0 / (1.0 + jnp.exp(-y)))[0]


def _tc_attention(e_d, w_big, wp_bd, rep, wout_bd, b_att_tile, b_out2):
    grid = BATCH // 128
    full = lambda shape: pl.BlockSpec(shape, lambda i: tuple(0 for _ in shape))
    return pl.pallas_call(
        _tc_body,
        grid=(grid,),
        in_specs=[
            pl.BlockSpec((NUM_PLANES, 128), lambda i: (0, i)),
            full((P_PAD, NUM_FIELDS)),
            full((P_PAD, NUM_FIELDS)),
            full((BT * EMBED_DIM, BT * ATT_VECTOR)),
            full((BT * ATT_VECTOR, BT)),
            full((BT, BT * EMBED_DIM)),
            full((BT * EMBED_DIM, BT)),
            full((1, BT * ATT_VECTOR)),
            full((1, 1)),
        ],
        out_specs=pl.BlockSpec((1, _SUB, BT), lambda i: (i, 0, 0)),
        out_shape=jax.ShapeDtypeStruct((grid, _SUB, BT), jnp.float32),
    )(e_d, jnp.asarray(_R), jnp.asarray(_C), w_big, wp_bd, rep, wout_bd,
      b_att_tile, b_out2)


def kernel(dense_inputs, sparse_inputs, embed_tables, W_att, b_att, w_proj,
           b_proj, W_out, b_out):
    del dense_inputs, b_proj
    t3 = jnp.transpose(embed_tables, (0, 2, 1))
    sidx = sparse_inputs.astype(jnp.int32).T

    e_d = _sc_gather(t3, sidx)

    eye = jnp.eye(BT, dtype=jnp.float32)
    w_big = jnp.kron(eye, W_att)
    wp_bd = jnp.kron(eye, w_proj)
    rep = jnp.kron(eye, jnp.ones((1, EMBED_DIM), jnp.float32))
    wout_bd = jnp.kron(eye, W_out)
    b_att_tile = jnp.tile(b_att, (BT,))[None, :]
    b_out2 = b_out.reshape(1, 1)

    out = _tc_attention(e_d, w_big, wp_bd, rep, wout_bd, b_att_tile, b_out2)
    return out.reshape(BATCH, 1)


# speedup vs baseline: 1.1114x; 1.1114x over previous
import functools
import itertools

import numpy as np
import jax
import jax.numpy as jnp
from jax import lax
from jax.experimental import pallas as pl
from jax.experimental.pallas import tpu as pltpu
from jax.experimental.pallas import tpu_sc as plsc

NUM_FIELDS = 26
VOCAB = 100000
EMBED_DIM = 16
ATT_VECTOR = 8
BATCH = 4096
NUM_PAIRS = NUM_FIELDS * (NUM_FIELDS - 1) // 2
P_PAD = 384
BT = 64
NUM_PLANES = NUM_FIELDS * EMBED_DIM

_row, _col = zip(*itertools.combinations(range(NUM_FIELDS), 2))
_R = np.zeros((P_PAD, NUM_FIELDS), np.float32)
_C = np.zeros((P_PAD, NUM_FIELDS), np.float32)
_R[np.arange(NUM_PAIRS), _row] = 1.0
_C[np.arange(NUM_PAIRS), _col] = 1.0


def _sc_gather(t3, sidx):
    info = plsc.get_sparse_core_info()
    nw = info.num_cores * info.num_subcores
    ppw = NUM_PLANES // nw
    mesh = plsc.VectorSubcoreMesh(core_axis_name="c", subcore_axis_name="s")

    @functools.partial(
        pl.kernel,
        out_type=jax.ShapeDtypeStruct((NUM_PLANES, BATCH), jnp.float32),
        mesh=mesh,
        scratch_types=[
            pltpu.VMEM((BATCH,), jnp.int32),
            pltpu.VMEM((ppw, BATCH), jnp.float32),
            pltpu.SemaphoreType.DMA,
            pltpu.SemaphoreType.DMA,
        ],
        compiler_params=pltpu.CompilerParams(use_tc_tiling_on_sc=False),
    )
    def gather_kernel(t_hbm, sidx_hbm, out_hbm, idx_v, vals_v, sem, gsem):
        wid = lax.axis_index("s") * info.num_cores + lax.axis_index("c")
        base = wid * ppw
        for j in range(ppw):
            plane = base + j
            f = plane // EMBED_DIM
            d = plane % EMBED_DIM
            pltpu.sync_copy(sidx_hbm.at[f], idx_v)
            pltpu.async_copy(t_hbm.at[f].at[d].at[idx_v], vals_v.at[j],
                             gsem).wait()
        pltpu.sync_copy(vals_v, out_hbm.at[pl.ds(base, ppw)])

    return gather_kernel(t3, sidx)


_SUB = 128 // BT


def _tc_body(e_ref, r_ref, c_ref, wbig_ref, wp_ref, rep_ref, wout_ref,
             batt_ref, bout_ref, o_ref):
    ed128 = e_ref[...]
    for t in range(_SUB):
        ed = ed128[:, t * BT:(t + 1) * BT]
        et = jnp.transpose(ed.reshape(NUM_FIELDS, EMBED_DIM, BT),
                           (0, 2, 1)).reshape(NUM_FIELDS, BT * EMBED_DIM)
        p = jnp.dot(r_ref[...], et, preferred_element_type=jnp.float32)
        q = jnp.dot(c_ref[...], et, preferred_element_type=jnp.float32)
        bi = p * q
        a1 = jnp.dot(bi.astype(jnp.bfloat16), wbig_ref[...],
                     preferred_element_type=jnp.float32)
        a1 = jnp.maximum(a1 + batt_ref[...], 0.0)
        logits = jnp.dot(a1, wp_ref[...], preferred_element_type=jnp.float32)
        pid = lax.broadcasted_iota(jnp.int32, logits.shape, 0)
        logits = jnp.where(pid < NUM_PAIRS, logits, -1e30)
        m = jnp.max(logits, axis=0, keepdims=True)
        ex = jnp.exp(logits - m)
        s = ex / jnp.sum(ex, axis=0, keepdims=True)
        s_exp = jnp.dot(s, rep_ref[...], preferred_element_type=jnp.float32)
        x = jnp.sum(bi * s_exp, axis=0, keepdims=True)
        y = jnp.dot(x, wout_ref[...], preferred_element_type=jnp.float32)
        y = y + bout_ref[...]
        o_ref[0, t] = (1.0 / (1.0 + jnp.exp(-y)))[0]


def _tc_attention(e_d, w_big, wp_bd, rep, wout_bd, b_att_tile, b_out2):
    grid = BATCH // 128
    full = lambda shape: pl.BlockSpec(shape, lambda i: tuple(0 for _ in shape))
    return pl.pallas_call(
        _tc_body,
        grid=(grid,),
        in_specs=[
            pl.BlockSpec((NUM_PLANES, 128), lambda i: (0, i)),
            full((P_PAD, NUM_FIELDS)),
            full((P_PAD, NUM_FIELDS)),
            full((BT * EMBED_DIM, BT * ATT_VECTOR)),
            full((BT * ATT_VECTOR, BT)),
            full((BT, BT * EMBED_DIM)),
            full((BT * EMBED_DIM, BT)),
            full((1, BT * ATT_VECTOR)),
            full((1, 1)),
        ],
        out_specs=pl.BlockSpec((1, _SUB, BT), lambda i: (i, 0, 0)),
        out_shape=jax.ShapeDtypeStruct((grid, _SUB, BT), jnp.float32),
    )(e_d, jnp.asarray(_R), jnp.asarray(_C), w_big, wp_bd, rep, wout_bd,
      b_att_tile, b_out2)


def kernel(dense_inputs, sparse_inputs, embed_tables, W_att, b_att, w_proj,
           b_proj, W_out, b_out):
    del dense_inputs, b_proj
    t3 = jnp.transpose(embed_tables, (0, 2, 1))
    sidx = sparse_inputs.astype(jnp.int32).T

    e_d = _sc_gather(t3, sidx)

    eye = jnp.eye(BT, dtype=jnp.float32)
    w_big = jnp.kron(eye, W_att).astype(jnp.bfloat16)
    wp_bd = jnp.kron(eye, w_proj)
    rep = jnp.kron(eye, jnp.ones((1, EMBED_DIM), jnp.float32))
    wout_bd = jnp.kron(eye, W_out)
    b_att_tile = jnp.tile(b_att, (BT,))[None, :]
    b_out2 = b_out.reshape(1, 1)

    out = _tc_attention(e_d, w_big, wp_bd, rep, wout_bd, b_att_tile, b_out2)
    return out.reshape(BATCH, 1)


# speedup vs baseline: 1.1785x; 1.0604x over previous
import functools
import itertools

import numpy as np
import jax
import jax.numpy as jnp
from jax import lax
from jax.experimental import pallas as pl
from jax.experimental.pallas import tpu as pltpu
from jax.experimental.pallas import tpu_sc as plsc

NUM_FIELDS = 26
VOCAB = 100000
EMBED_DIM = 16
ATT_VECTOR = 8
BATCH = 4096
NUM_PAIRS = NUM_FIELDS * (NUM_FIELDS - 1) // 2
P_PAD = 328
BT = 64
NUM_PLANES = NUM_FIELDS * EMBED_DIM

_row, _col = zip(*itertools.combinations(range(NUM_FIELDS), 2))
_R = np.zeros((P_PAD, NUM_FIELDS), np.float32)
_C = np.zeros((P_PAD, NUM_FIELDS), np.float32)
_R[np.arange(NUM_PAIRS), _row] = 1.0
_C[np.arange(NUM_PAIRS), _col] = 1.0


def _sc_gather(t3, sidx):
    info = plsc.get_sparse_core_info()
    nw = info.num_cores * info.num_subcores
    ppw = NUM_PLANES // nw
    mesh = plsc.VectorSubcoreMesh(core_axis_name="c", subcore_axis_name="s")

    @functools.partial(
        pl.kernel,
        out_type=jax.ShapeDtypeStruct((NUM_PLANES, BATCH), jnp.float32),
        mesh=mesh,
        scratch_types=[
            pltpu.VMEM((2, BATCH), jnp.int32),
            pltpu.VMEM((ppw, BATCH), jnp.float32),
            pltpu.SemaphoreType.DMA,
            pltpu.SemaphoreType.DMA,
        ],
        compiler_params=pltpu.CompilerParams(use_tc_tiling_on_sc=False),
    )
    def gather_kernel(t_hbm, sidx_hbm, out_hbm, idx_v, vals_v, sem, gsem):
        wid = lax.axis_index("s") * info.num_cores + lax.axis_index("c")
        base = wid * ppw
        f0 = base // EMBED_DIM
        f1 = (base + ppw - 1) // EMBED_DIM
        pltpu.sync_copy(sidx_hbm.at[f0], idx_v.at[0])
        pltpu.sync_copy(sidx_hbm.at[f1], idx_v.at[1])
        copies = []
        for j in range(ppw):
            plane = base + j
            f = plane // EMBED_DIM
            d = plane % EMBED_DIM
            copies.append(pltpu.async_copy(
                t_hbm.at[f].at[d].at[idx_v.at[f - f0]], vals_v.at[j], gsem))
        for c in copies:
            c.wait()
        pltpu.sync_copy(vals_v, out_hbm.at[pl.ds(base, ppw)])

    return gather_kernel(t3, sidx)


_SUB = 128 // BT


def _tc_body(e_ref, r_ref, c_ref, wbig_ref, wp_ref, rep_ref, wout_ref,
             batt_ref, bout_ref, o_ref):
    ed128 = e_ref[...]
    for t in range(_SUB):
        ed = ed128[:, t * BT:(t + 1) * BT]
        et = jnp.transpose(ed.reshape(NUM_FIELDS, EMBED_DIM, BT),
                           (0, 2, 1)).reshape(NUM_FIELDS, BT * EMBED_DIM)
        p = jnp.dot(r_ref[...], et, preferred_element_type=jnp.float32)
        q = jnp.dot(c_ref[...], et, preferred_element_type=jnp.float32)
        bi = p * q
        a1 = jnp.dot(bi.astype(jnp.bfloat16), wbig_ref[...],
                     preferred_element_type=jnp.float32)
        a1 = jnp.maximum(a1 + batt_ref[...], 0.0)
        logits = jnp.dot(a1, wp_ref[...], preferred_element_type=jnp.float32)
        pid = lax.broadcasted_iota(jnp.int32, logits.shape, 0)
        logits = jnp.where(pid < NUM_PAIRS, logits, -1e30)
        m = jnp.max(logits, axis=0, keepdims=True)
        ex = jnp.exp(logits - m)
        s = ex / jnp.sum(ex, axis=0, keepdims=True)
        s_exp = jnp.dot(s, rep_ref[...], preferred_element_type=jnp.float32)
        x = jnp.sum(bi * s_exp, axis=0, keepdims=True)
        y = jnp.dot(x, wout_ref[...], preferred_element_type=jnp.float32)
        y = y + bout_ref[...]
        o_ref[0, t] = (1.0 / (1.0 + jnp.exp(-y)))[0]


def _tc_attention(e_d, w_big, wp_bd, rep, wout_bd, b_att_tile, b_out2):
    grid = BATCH // 128
    full = lambda shape: pl.BlockSpec(shape, lambda i: tuple(0 for _ in shape))
    return pl.pallas_call(
        _tc_body,
        grid=(grid,),
        in_specs=[
            pl.BlockSpec((NUM_PLANES, 128), lambda i: (0, i)),
            full((P_PAD, NUM_FIELDS)),
            full((P_PAD, NUM_FIELDS)),
            full((BT * EMBED_DIM, BT * ATT_VECTOR)),
            full((BT * ATT_VECTOR, BT)),
            full((BT, BT * EMBED_DIM)),
            full((BT * EMBED_DIM, BT)),
            full((1, BT * ATT_VECTOR)),
            full((1, 1)),
        ],
        out_specs=pl.BlockSpec((1, _SUB, BT), lambda i: (i, 0, 0)),
        out_shape=jax.ShapeDtypeStruct((grid, _SUB, BT), jnp.float32),
    )(e_d, jnp.asarray(_R), jnp.asarray(_C), w_big, wp_bd, rep, wout_bd,
      b_att_tile, b_out2)


def kernel(dense_inputs, sparse_inputs, embed_tables, W_att, b_att, w_proj,
           b_proj, W_out, b_out):
    del dense_inputs, b_proj
    t3 = jnp.transpose(embed_tables, (0, 2, 1))
    sidx = sparse_inputs.astype(jnp.int32).T

    e_d = _sc_gather(t3, sidx)

    eye = jnp.eye(BT, dtype=jnp.float32)
    w_big = jnp.kron(eye, W_att).astype(jnp.bfloat16)
    wp_bd = jnp.kron(eye, w_proj)
    rep = jnp.kron(eye, jnp.ones((1, EMBED_DIM), jnp.float32))
    wout_bd = jnp.kron(eye, W_out)
    b_att_tile = jnp.tile(b_att, (BT,))[None, :]
    b_out2 = b_out.reshape(1, 1)

    out = _tc_attention(e_d, w_big, wp_bd, rep, wout_bd, b_att_tile, b_out2)
    return out.reshape(BATCH, 1)


# speedup vs baseline: 1.2499x; 1.0606x over previous
import functools
import itertools

import numpy as np
import jax
import jax.numpy as jnp
from jax import lax
from jax.experimental import pallas as pl
from jax.experimental.pallas import tpu as pltpu
from jax.experimental.pallas import tpu_sc as plsc

NUM_FIELDS = 26
VOCAB = 100000
EMBED_DIM = 16
ATT_VECTOR = 8
BATCH = 4096
NUM_PAIRS = NUM_FIELDS * (NUM_FIELDS - 1) // 2
P_PAD = 328
BT = 64
NUM_PLANES = NUM_FIELDS * EMBED_DIM

_row, _col = zip(*itertools.combinations(range(NUM_FIELDS), 2))
_R = np.zeros((P_PAD, NUM_FIELDS), np.float32)
_C = np.zeros((P_PAD, NUM_FIELDS), np.float32)
_R[np.arange(NUM_PAIRS), _row] = 1.0
_C[np.arange(NUM_PAIRS), _col] = 1.0


def _sc_gather(t3, sidx):
    info = plsc.get_sparse_core_info()
    nw = info.num_cores * info.num_subcores
    ppw = NUM_PLANES // nw
    nb = sidx.shape[1]
    mesh = plsc.VectorSubcoreMesh(core_axis_name="c", subcore_axis_name="s")

    @functools.partial(
        pl.kernel,
        out_type=jax.ShapeDtypeStruct((NUM_PLANES, nb), jnp.float32),
        mesh=mesh,
        scratch_types=[
            pltpu.VMEM((2, nb), jnp.int32),
            pltpu.VMEM((ppw, nb), jnp.float32),
            pltpu.SemaphoreType.DMA,
            pltpu.SemaphoreType.DMA,
        ],
        compiler_params=pltpu.CompilerParams(use_tc_tiling_on_sc=False),
    )
    def gather_kernel(t_hbm, sidx_hbm, out_hbm, idx_v, vals_v, sem, gsem):
        wid = lax.axis_index("s") * info.num_cores + lax.axis_index("c")
        base = wid * ppw
        f0 = base // EMBED_DIM
        f1 = (base + ppw - 1) // EMBED_DIM
        pltpu.sync_copy(sidx_hbm.at[f0], idx_v.at[0])
        pltpu.sync_copy(sidx_hbm.at[f1], idx_v.at[1])
        copies = []
        for j in range(ppw):
            plane = base + j
            f = plane // EMBED_DIM
            d = plane % EMBED_DIM
            copies.append(pltpu.async_copy(
                t_hbm.at[f].at[d].at[idx_v.at[f - f0]], vals_v.at[j], gsem))
        for c in copies:
            c.wait()
        pltpu.sync_copy(vals_v, out_hbm.at[pl.ds(base, ppw)])

    return gather_kernel(t3, sidx)


_SUB = 128 // BT


def _tc_body(e_ref, r_ref, c_ref, wbig_ref, wp_ref, rep_ref, wout_ref,
             batt_ref, bout_ref, o_ref):
    ed128 = e_ref[...]
    for t in range(_SUB):
        ed = ed128[:, t * BT:(t + 1) * BT]
        et = jnp.transpose(ed.reshape(NUM_FIELDS, EMBED_DIM, BT),
                           (0, 2, 1)).reshape(NUM_FIELDS, BT * EMBED_DIM)
        p = jnp.dot(r_ref[...], et, preferred_element_type=jnp.float32)
        q = jnp.dot(c_ref[...], et, preferred_element_type=jnp.float32)
        bi = p * q
        a1 = jnp.dot(bi.astype(jnp.bfloat16), wbig_ref[...],
                     preferred_element_type=jnp.float32)
        a1 = jnp.maximum(a1 + batt_ref[...], 0.0)
        logits = jnp.dot(a1, wp_ref[...], preferred_element_type=jnp.float32)
        pid = lax.broadcasted_iota(jnp.int32, logits.shape, 0)
        logits = jnp.where(pid < NUM_PAIRS, logits, -1e30)
        m = jnp.max(logits, axis=0, keepdims=True)
        ex = jnp.exp(logits - m)
        s = ex / jnp.sum(ex, axis=0, keepdims=True)
        s_exp = jnp.dot(s, rep_ref[...], preferred_element_type=jnp.float32)
        x = jnp.sum(bi * s_exp, axis=0, keepdims=True)
        y = jnp.dot(x, wout_ref[...], preferred_element_type=jnp.float32)
        y = y + bout_ref[...]
        o_ref[0, t] = (1.0 / (1.0 + jnp.exp(-y)))[0]


def _tc_attention(e_d, w_big, wp_bd, rep, wout_bd, b_att_tile, b_out2):
    grid = e_d.shape[1] // 128
    full = lambda shape: pl.BlockSpec(shape, lambda i: tuple(0 for _ in shape))
    return pl.pallas_call(
        _tc_body,
        grid=(grid,),
        in_specs=[
            pl.BlockSpec((NUM_PLANES, 128), lambda i: (0, i)),
            full((P_PAD, NUM_FIELDS)),
            full((P_PAD, NUM_FIELDS)),
            full((BT * EMBED_DIM, BT * ATT_VECTOR)),
            full((BT * ATT_VECTOR, BT)),
            full((BT, BT * EMBED_DIM)),
            full((BT * EMBED_DIM, BT)),
            full((1, BT * ATT_VECTOR)),
            full((1, 1)),
        ],
        out_specs=pl.BlockSpec((1, _SUB, BT), lambda i: (i, 0, 0)),
        out_shape=jax.ShapeDtypeStruct((grid, _SUB, BT), jnp.float32),
    )(e_d, jnp.asarray(_R), jnp.asarray(_C), w_big, wp_bd, rep, wout_bd,
      b_att_tile, b_out2)


def kernel(dense_inputs, sparse_inputs, embed_tables, W_att, b_att, w_proj,
           b_proj, W_out, b_out):
    del dense_inputs, b_proj
    t3 = jnp.transpose(embed_tables, (0, 2, 1))
    sidx = sparse_inputs.astype(jnp.int32).T


    eye = jnp.eye(BT, dtype=jnp.float32)
    w_big = jnp.kron(eye, W_att).astype(jnp.bfloat16)
    wp_bd = jnp.kron(eye, w_proj)
    rep = jnp.kron(eye, jnp.ones((1, EMBED_DIM), jnp.float32))
    wout_bd = jnp.kron(eye, W_out)
    b_att_tile = jnp.tile(b_att, (BT,))[None, :]
    b_out2 = b_out.reshape(1, 1)

    h = BATCH // 2
    outs = []
    for k in range(2):
        e_h = _sc_gather(t3, sidx[:, k * h:(k + 1) * h])
        outs.append(_tc_attention(e_h, w_big, wp_bd, rep, wout_bd,
                                  b_att_tile, b_out2).reshape(h, 1))
    return jnp.concatenate(outs, axis=0)
